# Initial kernel scaffold; baseline (speedup 1.0000x reference)
import jax
import jax.numpy as jnp
from jax.experimental import pallas as pl


def kernel(u_hat, b, routing_num):
    raise NotImplementedError("write your pallas kernel here")


# speedup vs baseline: 12.0766x; 12.0766x over previous
import jax
import jax.numpy as jnp
from jax import lax
from jax.experimental import pallas as pl
from jax.experimental.pallas import tpu as pltpu
from jax.experimental.pallas import tpu_sc as plsc

_I = 65536
_J = 64
_F = 16
_JF = _J * _F
_NC = 2
_NS = 16
_NW = _NC * _NS
_IPW = _I // _NW
_C = 32
_NCHUNK = _IPW // _C


def _sc_pass_body(u_hbm, b_hbm, w_hbm, out_hbm,
                  ub0, ub1, bb0, bb1, wvm, sv, lscr, cscr,
                  semu0, semu1, semb0, semb1):
    wid = lax.axis_index("s") * _NC + lax.axis_index("c")
    wbase = wid * _IPW

    ubufs = (ub0, ub1)
    bbufs = (bb0, bb1)
    semus = (semu0, semu1)
    sembs = (semb0, semb1)

    lane = jax.lax.iota(jnp.int32, 16)

    pltpu.sync_copy(w_hbm, wvm)

    def _zero(j, _):
        sv[j, :] = jnp.zeros((_F,), jnp.float32)
        return 0
    lax.fori_loop(0, _J, _zero, 0, unroll=8)

    def _start(g, ph):
        base = wbase + g * _C
        pltpu.async_copy(u_hbm.at[pl.ds(base, _C)], ubufs[ph], semus[ph])
        pltpu.async_copy(b_hbm.at[pl.ds(base, _C)], bbufs[ph], sembs[ph])

    def _wait(ph):
        pltpu.make_async_copy(u_hbm.at[pl.ds(0, _C)], ubufs[ph], semus[ph]).wait()
        pltpu.make_async_copy(b_hbm.at[pl.ds(0, _C)], bbufs[ph], sembs[ph]).wait()

    _start(0, 0)

    def _process_chunk(ub, bb):
        def _jdot(j, _):
            col0 = j * _F
            wv = wvm[j, :]
            for cb in range(_C // 16):
                idx0 = cb * 16 + lane
                acc = jnp.zeros((16,), jnp.float32)
                for f in range(_F):
                    g = plsc.load_gather(
                        ub, [idx0, jnp.full((16,), col0 + f, jnp.int32)])
                    acc = acc + g * wv[f]
                lscr[j, pl.ds(cb * 16, 16)] = acc
            return 0
        lax.fori_loop(0, _J, _jdot, 0)

        def _smax(ci, _):
            ls = [plsc.load_gather(lscr, [jg * 16 + lane,
                                          jnp.full((16,), ci, jnp.int32)])
                  + bb[ci, pl.ds(jg * 16, 16)]
                  for jg in range(4)]
            mv = jnp.maximum(jnp.maximum(ls[0], ls[1]),
                             jnp.maximum(ls[2], ls[3]))
            m = jnp.max(mv)
            es = [jnp.exp(l - m) for l in ls]
            z = (jnp.sum(es[0]) + jnp.sum(es[1])
                 + jnp.sum(es[2]) + jnp.sum(es[3]))
            rz = jnp.full((16,), 1.0, jnp.float32) / jnp.full((16,), z, jnp.float32)
            for jg in range(4):
                cscr[ci, pl.ds(jg * 16, 16)] = es[jg] * rz
            return 0
        lax.fori_loop(0, _C, _smax, 0)

        def _imsg(ci, _):
            cvecs = [cscr[ci, pl.ds(jg * 16, 16)] for jg in range(4)]
            for j in range(_J):
                cs = cvecs[j // 16][j % 16]
                plsc.addupdate(sv.at[j, :], ub[ci, pl.ds(j * _F, _F)] * cs)
            return 0
        lax.fori_loop(0, _C, _imsg, 0)

    def _chunk_loop(g2, _):
        for ph in range(2):
            g = g2 * 2 + ph

            @pl.when(g + 1 < _NCHUNK)
            def _():
                _start(g + 1, 1 - ph)

            _wait(ph)
            _process_chunk(ubufs[ph], bbufs[ph])
        return 0

    lax.fori_loop(0, _NCHUNK // 2, _chunk_loop, 0)

    pltpu.sync_copy(sv, out_hbm.at[wid])


_sc_pass = pl.kernel(
    _sc_pass_body,
    out_type=jax.ShapeDtypeStruct((_NW, _J, _F), jnp.float32),
    mesh=plsc.VectorSubcoreMesh(core_axis_name="c", subcore_axis_name="s",
                                num_cores=_NC, num_subcores=_NS),
    scratch_types=[
        pltpu.VMEM((_C, _JF), jnp.float32),
        pltpu.VMEM((_C, _JF), jnp.float32),
        pltpu.VMEM((_C, _J), jnp.float32),
        pltpu.VMEM((_C, _J), jnp.float32),
        pltpu.VMEM((_J, _F), jnp.float32),
        pltpu.VMEM((_J, _F), jnp.float32),
        pltpu.VMEM((_J, _C), jnp.float32),
        pltpu.VMEM((_C, _J), jnp.float32),
        pltpu.SemaphoreType.DMA,
        pltpu.SemaphoreType.DMA,
        pltpu.SemaphoreType.DMA,
        pltpu.SemaphoreType.DMA,
    ],
    compiler_params=pltpu.CompilerParams(use_tc_tiling_on_sc=False,
                                         needs_layout_passes=False),
    name="dgl_routing_sc_pass",
)


def _combine_body(sp_ref, w_ref, v_ref, wn_ref):
    s = jnp.sum(sp_ref[...], axis=0)
    sq = jnp.sum(s * s, axis=1, keepdims=True)
    v = sq / (1.0 + sq) * (s / jnp.sqrt(sq))
    v_ref[...] = v
    wn_ref[...] = w_ref[...] + v


_combine = pl.pallas_call(
    _combine_body,
    out_shape=(jax.ShapeDtypeStruct((_J, _F), jnp.float32),
               jax.ShapeDtypeStruct((_J, _F), jnp.float32)),
    name="dgl_routing_squash",
)


def kernel(u_hat, b, routing_num):
    u2 = u_hat.reshape(_I, _JF)
    b2 = b.reshape(_I, _J)

    def body(_, carry):
        v, w = carry
        sp = _sc_pass(u2, b2, w)
        v, w = _combine(sp, w)
        return v, w

    v0 = jnp.zeros((_J, _F), jnp.float32)
    w0 = jnp.zeros((_J, _F), jnp.float32)
    v, _ = lax.fori_loop(0, routing_num, body, (v0, w0))
    return v
